# baseline (speedup 1.0000x reference)
import functools

import jax
import jax.numpy as jnp
from jax import lax
from jax.experimental import pallas as pl
from jax.experimental.pallas import tpu as pltpu
from jax.experimental.pallas import tpu_sc as plsc

N = 10000
E = 320000
F = 128
NB = 1024

NC = 2
NS = 16
NW = NC * NS
L = 16

EPW = E // NW
EPP = 10080
NPAD = N + 16
C1 = 120
NJ1 = EPP // C1
NCH1 = 12
NSUP1 = NJ1 // NCH1
C2 = 80
NJ2 = EPW // C2
NG2 = C2 // L

RPT = N // NS
DEGP = 10240
DSTRIPE = DEGP // NS

_mesh = plsc.VectorSubcoreMesh(
    core_axis_name="c", subcore_axis_name="s", num_cores=NC, num_subcores=NS)


def _wid():
    return lax.axis_index("s") * NC + lax.axis_index("c")


@functools.partial(
    pl.kernel,
    out_type=jax.ShapeDtypeStruct((NC, DEGP), jnp.float32),
    mesh=_mesh,
    compiler_params=pltpu.CompilerParams(use_tc_tiling_on_sc=False, needs_layout_passes=False),
    scratch_types=[
        pltpu.VMEM((NJ1, C1), jnp.int32),
        pltpu.VMEM((128,), jnp.float32),
        pltpu.VMEM((DSTRIPE,), jnp.float32),
        pltpu.VMEM_SHARED((DEGP,), jnp.float32),
    ],
)
def _deg_kernel(dst_hbm, out_hbm, dstl, ones_v, strip_v, deg_sh):
    c = lax.axis_index("c")
    s = lax.axis_index("s")
    w = _wid()

    def fill(i, _):
        strip_v[pl.ds(i * L, L)] = jnp.zeros((L,), jnp.float32)
        return 0
    lax.fori_loop(0, DSTRIPE // L, fill, 0)

    def fill1(i, _):
        ones_v[pl.ds(i * L, L)] = jnp.ones((L,), jnp.float32)
        return 0
    lax.fori_loop(0, 8, fill1, 0)

    pltpu.sync_copy(strip_v, deg_sh.at[pl.ds(s * DSTRIPE, DSTRIPE)])
    plsc.subcore_barrier()

    pltpu.sync_copy(dst_hbm.at[w], dstl)

    def chunk(j, _):
        pltpu.sync_copy(ones_v.at[pl.ds(0, C1)], deg_sh.at[dstl.at[j]],
                        add=True)
        return 0
    lax.fori_loop(0, NJ1, chunk, 0)

    plsc.subcore_barrier()
    pltpu.sync_copy(deg_sh.at[pl.ds(s * DSTRIPE, DSTRIPE)], strip_v)
    pltpu.sync_copy(strip_v, out_hbm.at[c, pl.ds(s * DSTRIPE, DSTRIPE)])


@functools.partial(
    pl.kernel,
    out_type=jax.ShapeDtypeStruct((NC, N, F), jnp.float32),
    mesh=_mesh,
    compiler_params=pltpu.CompilerParams(use_tc_tiling_on_sc=False, needs_layout_passes=False),
    scratch_types=[
        pltpu.VMEM((NCH1, C1), jnp.int32),
        pltpu.VMEM((NCH1, C1), jnp.int32),
        pltpu.VMEM((C1, F), jnp.float32),
        pltpu.VMEM((C1, F), jnp.float32),
        pltpu.VMEM((C1, F), jnp.float32),
        pltpu.VMEM_SHARED((NPAD, F), jnp.float32),
        pltpu.SemaphoreType.DMA,
        pltpu.SemaphoreType.DMA,
        pltpu.SemaphoreType.DMA,
        pltpu.SemaphoreType.DMA,
    ],
)
def _conv1_kernel(y_hbm, src_hbm, dst_hbm, zro_hbm, out_hbm, srcl, dstl,
                  rows0, rows1, rows2, agg_sh, sg, ss0, ss1, ss2):
    c = lax.axis_index("c")
    s = lax.axis_index("s")
    w = _wid()
    rows = (rows0, rows1, rows2)
    ss = (ss0, ss1, ss2)

    pltpu.sync_copy(zro_hbm, agg_sh.at[pl.ds(s * RPT, RPT)])
    plsc.subcore_barrier()

    def sdrain(k):
        pltpu.make_async_copy(rows[k], agg_sh.at[dstl.at[0]], ss[k]).wait()

    def sup(so, _):
        @pl.when(so > 0)
        def _():
            sdrain((NCH1 - 2) % 3)
            sdrain((NCH1 - 1) % 3)
        pltpu.sync_copy(src_hbm.at[w, pl.ds(so * NCH1, NCH1)], srcl)
        pltpu.sync_copy(dst_hbm.at[w, pl.ds(so * NCH1, NCH1)], dstl)
        pltpu.async_copy(y_hbm.at[srcl.at[0]], rows0, sg)

        def triple(tr, _):
            for k in range(3):
                j = tr * 3 + k
                pltpu.make_async_copy(y_hbm.at[srcl.at[j]], rows[k],
                                      sg).wait()

                @pl.when(j >= 2)
                def _():
                    sdrain((k + 1) % 3)

                @pl.when(j + 1 < NCH1)
                def _():
                    pltpu.async_copy(y_hbm.at[srcl.at[j + 1]],
                                     rows[(k + 1) % 3], sg)
                pltpu.async_copy(rows[k], agg_sh.at[dstl.at[j]], ss[k],
                                 add=True)
            return 0
        lax.fori_loop(0, NCH1 // 3, triple, 0)
        return 0
    lax.fori_loop(0, NSUP1, sup, 0)

    sdrain((NCH1 - 2) % 3)
    sdrain((NCH1 - 1) % 3)
    plsc.subcore_barrier()
    pltpu.sync_copy(agg_sh.at[pl.ds(s * RPT, RPT)],
                    out_hbm.at[c, pl.ds(s * RPT, RPT)])


@functools.partial(
    pl.kernel,
    out_type=jax.ShapeDtypeStruct((NC, N, F), jnp.float32),
    mesh=_mesh,
    compiler_params=pltpu.CompilerParams(use_tc_tiling_on_sc=False, needs_layout_passes=False),
    scratch_types=[
        pltpu.VMEM((NJ2 // 5, C2), jnp.int32),
        pltpu.VMEM((NJ2 // 5, C2), jnp.int32),
        pltpu.VMEM((C2, F), jnp.float32),
        pltpu.VMEM((C2, F), jnp.float32),
        pltpu.VMEM((C2, F), jnp.float32),
        pltpu.VMEM((N,), jnp.float32),
        pltpu.VMEM((L,), jnp.float32),
        pltpu.VMEM_SHARED((N, F), jnp.float32),
        pltpu.SemaphoreType.DMA,
        pltpu.SemaphoreType.DMA,
    ],
)
def _conv2_kernel(h_hbm, src_hbm, dst_hbm, isd_hbm, gam_hbm, out_hbm,
                  srcl, dstl, sbuf, dbuf, mbuf, isd_v, w_v, agg_sh, sem,
                  sem2):
    c = lax.axis_index("c")
    s = lax.axis_index("s")
    w = _wid()
    nsup = NJ2 // 5

    def fill(i, _):
        sbuf[i // (F // L), pl.ds((i % (F // L)) * L, L)] = (
            jnp.zeros((L,), jnp.float32))
        return 0
    lax.fori_loop(0, C2 * (F // L), fill, 0)
    for t in range(RPT // C2):
        pltpu.sync_copy(sbuf, agg_sh.at[pl.ds(s * RPT + t * C2, C2)])
    pltpu.sync_copy(sbuf.at[pl.ds(0, RPT - (RPT // C2) * C2)],
                    agg_sh.at[pl.ds(s * RPT + (RPT // C2) * C2,
                                    RPT - (RPT // C2) * C2)])
    plsc.subcore_barrier()

    pltpu.sync_copy(isd_hbm, isd_v)
    pltpu.sync_copy(gam_hbm, w_v)
    gam = w_v[...]

    def sup(so, _):
        @pl.when(so > 0)
        def _():
            pltpu.make_async_copy(mbuf, agg_sh.at[dstl.at[nsup - 1]],
                                  sem2).wait()
        pltpu.sync_copy(src_hbm.at[w, pl.ds(so * nsup, nsup)], srcl)
        pltpu.sync_copy(dst_hbm.at[w, pl.ds(so * nsup, nsup)], dstl)

        def chunk(j, _):
            cp_s = pltpu.async_copy(h_hbm.at[srcl.at[j]], sbuf, sem)
            cp_d = pltpu.async_copy(h_hbm.at[dstl.at[j]], dbuf, sem)

            @pl.when(j > 0)
            def _():
                pltpu.make_async_copy(mbuf, agg_sh.at[dstl.at[j]],
                                      sem2).wait()
            cp_s.wait()
            cp_d.wait()

            @plsc.parallel_loop(0, NG2, 1, unroll=2)
            def group(g):
                src_v = srcl[j, pl.ds(g * L, L)]
                isd_s = plsc.load_gather(isd_v, [src_v])
                for el in range(L):
                    e = g * L + el
                    sv = [sbuf[e, pl.ds(r * L, L)] for r in range(F // L)]
                    sq = [None] * (F // L)
                    for r in range(F // L):
                        d = sv[r] - dbuf[e, pl.ds(r * L, L)]
                        sq[r] = d * d
                    acc = (((sq[0] + sq[1]) + (sq[2] + sq[3]))
                           + ((sq[4] + sq[5]) + (sq[6] + sq[7])))
                    d2 = lax.broadcast(jnp.sum(acc), (L,))
                    wb = jnp.exp(-gam * d2) * lax.broadcast(isd_s[el], (L,))
                    for r in range(F // L):
                        mbuf[e, pl.ds(r * L, L)] = sv[r] * wb
            pltpu.async_copy(mbuf, agg_sh.at[dstl.at[j]], sem2, add=True)
            return 0
        lax.fori_loop(0, nsup, chunk, 0)
        return 0
    lax.fori_loop(0, 5, sup, 0)

    pltpu.make_async_copy(mbuf, agg_sh.at[dstl.at[nsup - 1]], sem2).wait()
    plsc.subcore_barrier()
    for t in range(RPT // C2):
        pltpu.sync_copy(agg_sh.at[pl.ds(s * RPT + t * C2, C2)], sbuf)
        pltpu.sync_copy(sbuf, out_hbm.at[c, pl.ds(s * RPT + t * C2, C2)])
    rem = RPT - (RPT // C2) * C2
    pltpu.sync_copy(agg_sh.at[pl.ds(s * RPT + (RPT // C2) * C2, rem)],
                    sbuf.at[pl.ds(0, rem)])
    pltpu.sync_copy(sbuf.at[pl.ds(0, rem)],
                    out_hbm.at[c, pl.ds(s * RPT + (RPT // C2) * C2, rem)])


@functools.partial(
    pl.kernel,
    out_type=jax.ShapeDtypeStruct((NB, F), jnp.float32),
    mesh=_mesh,
    compiler_params=pltpu.CompilerParams(use_tc_tiling_on_sc=False, needs_layout_passes=False),
    scratch_types=[
        pltpu.VMEM((NB // NW,), jnp.int32),
        pltpu.VMEM((NB // NW, F), jnp.float32),
        pltpu.SemaphoreType.DMA,
    ],
)
def _gather_kernel(xh_hbm, idx_hbm, out_hbm, idx_v, rows, sem):
    w = _wid()
    bpw = NB // NW
    pltpu.sync_copy(idx_hbm.at[w], idx_v)
    pltpu.async_copy(xh_hbm.at[idx_v], rows, sem).wait()
    pltpu.sync_copy(rows, out_hbm.at[pl.ds(w * bpw, bpw)])


def _pre_body(dp_ref, x_ref, isd_ref, y_ref):
    d = dp_ref[:, 0:1] + dp_ref[:, 1:2] + 1.0
    isd = lax.rsqrt(d)
    isd_ref[...] = isd
    y_ref[0:N, :] = x_ref[...] * isd
    y_ref[N:NPAD, :] = jnp.zeros((NPAD - N, F), jnp.float32)


def _dense1_body(x_ref, ap_ref, isd_ref, k2_ref, w_ref, b_ref, bnw_ref,
                 bnb_ref, h1_ref, helm_ref):
    k2 = k2_ref[0, 0]
    agg = isd_ref[...] * (ap_ref[0] + ap_ref[1])
    x = x_ref[...]
    diff = agg - k2 * x
    helm_ref[...] = jnp.full((1, 1), 0.5, jnp.float32) * jnp.mean(
        diff * diff, keepdims=True)
    h = k2 * x + agg
    o = jnp.dot(h, w_ref[...], preferred_element_type=jnp.float32) + b_ref[...]
    m = jnp.mean(o, axis=0, keepdims=True)
    cdev = o - m
    v = jnp.mean(cdev * cdev, axis=0, keepdims=True)
    h1_ref[...] = jnp.tanh(cdev / jnp.sqrt(v + 1e-5) * bnw_ref[...]
                           + bnb_ref[...])


def _dense2_body(x_ref, ap_ref, isd_ref, k2_ref, w_ref, b_ref, bnw_ref,
                 bnb_ref, out_ref):
    k2 = k2_ref[0, 0]
    agg = isd_ref[...] * (ap_ref[0] + ap_ref[1])
    h = k2 * x_ref[...] + agg
    o = jnp.dot(h, w_ref[...], preferred_element_type=jnp.float32) + b_ref[...]
    m = jnp.mean(o, axis=0, keepdims=True)
    cdev = o - m
    v = jnp.mean(cdev * cdev, axis=0, keepdims=True)
    xh = jnp.tanh(cdev / jnp.sqrt(v + 1e-5) * bnw_ref[...] + bnb_ref[...])
    m2 = jnp.max(xh, axis=1, keepdims=True)
    ex = jnp.exp(xh - m2)
    lse = jnp.log(jnp.sum(ex, axis=1, keepdims=True))
    out_ref[...] = xh - m2 - lse


def kernel(features, edge_index, batch_nodes, device, W1, b1, k2_1, gamma_1,
           bn1_w, bn1_b, W2, b2, k2_2, gamma_2, bn2_w, bn2_b):
    x = features
    src = edge_index[0]
    dst = edge_index[1]
    epad = jnp.full((NW, EPP - EPW), N, jnp.int32)
    src3a = jnp.concatenate([src.reshape(NW, EPW), epad],
                            axis=1).reshape(NW, NJ1, C1)
    dst3a = jnp.concatenate([dst.reshape(NW, EPW), epad],
                            axis=1).reshape(NW, NJ1, C1)
    src3b = src.reshape(NW, NJ2, C2)
    dst3b = dst.reshape(NW, NJ2, C2)

    degp = _deg_kernel(dst3a)
    dpT = degp[:, :N].T

    isd, y = pl.pallas_call(
        _pre_body,
        out_shape=[jax.ShapeDtypeStruct((N, 1), jnp.float32),
                   jax.ShapeDtypeStruct((NPAD, F), jnp.float32)],
    )(dpT, x)

    zro = jnp.zeros((RPT, F), jnp.float32)
    aggp1 = _conv1_kernel(y, src3a, dst3a, zro)

    h1, helm = pl.pallas_call(
        _dense1_body,
        out_shape=[jax.ShapeDtypeStruct((N, F), jnp.float32),
                   jax.ShapeDtypeStruct((1, 1), jnp.float32)],
    )(x, aggp1, isd, k2_1.reshape(1, 1), W1, b1.reshape(1, F),
      bn1_w.reshape(1, F), bn1_b.reshape(1, F))

    gam16 = jnp.full((L,), gamma_2, jnp.float32)
    aggp2 = _conv2_kernel(h1, src3b, dst3b, isd.reshape(N), gam16)

    xh = pl.pallas_call(
        _dense2_body,
        out_shape=jax.ShapeDtypeStruct((N, F), jnp.float32),
    )(h1, aggp2, isd, k2_2.reshape(1, 1), W2, b2.reshape(1, F),
      bn2_w.reshape(1, F), bn2_b.reshape(1, F))

    logits = _gather_kernel(xh, batch_nodes.reshape(NW, NB // NW))

    return logits, helm.reshape(())


# speedup vs baseline: 1.1659x; 1.1659x over previous
import functools

import jax
import jax.numpy as jnp
from jax import lax
from jax.experimental import pallas as pl
from jax.experimental.pallas import tpu as pltpu
from jax.experimental.pallas import tpu_sc as plsc

N = 10000
E = 320000
F = 128
NB = 1024

NC = 2
NS = 16
NW = NC * NS
L = 16

EPW = E // NW
C1 = 125
NJ1 = EPW // C1
C2 = 80
NJ2 = EPW // C2
NG2 = C2 // L

RPT = N // NS
DEGP = 10240
DSTRIPE = DEGP // NS

_mesh = plsc.VectorSubcoreMesh(
    core_axis_name="c", subcore_axis_name="s", num_cores=NC, num_subcores=NS)


def _wid():
    return lax.axis_index("s") * NC + lax.axis_index("c")


@functools.partial(
    pl.kernel,
    out_type=jax.ShapeDtypeStruct((NC, DEGP), jnp.float32),
    mesh=_mesh,
    compiler_params=pltpu.CompilerParams(use_tc_tiling_on_sc=False, needs_layout_passes=False),
    scratch_types=[
        pltpu.VMEM((NJ1, C1), jnp.int32),
        pltpu.VMEM((C1 + 3,), jnp.float32),
        pltpu.VMEM((DSTRIPE,), jnp.float32),
        pltpu.VMEM_SHARED((DEGP,), jnp.float32),
    ],
)
def _deg_kernel(dst_hbm, out_hbm, dstl, ones_v, strip_v, deg_sh):
    c = lax.axis_index("c")
    s = lax.axis_index("s")
    w = _wid()

    def fill(i, _):
        strip_v[pl.ds(i * L, L)] = jnp.zeros((L,), jnp.float32)
        return 0
    lax.fori_loop(0, DSTRIPE // L, fill, 0)

    def fill1(i, _):
        ones_v[pl.ds(i * L, L)] = jnp.ones((L,), jnp.float32)
        return 0
    lax.fori_loop(0, (C1 + 3) // L, fill1, 0)

    pltpu.sync_copy(strip_v, deg_sh.at[pl.ds(s * DSTRIPE, DSTRIPE)])
    plsc.subcore_barrier()

    pltpu.sync_copy(dst_hbm.at[w], dstl)

    def chunk(j, _):
        pltpu.sync_copy(ones_v.at[pl.ds(0, C1)], deg_sh.at[dstl.at[j]],
                        add=True)
        return 0
    lax.fori_loop(0, NJ1, chunk, 0)

    plsc.subcore_barrier()
    pltpu.sync_copy(deg_sh.at[pl.ds(s * DSTRIPE, DSTRIPE)], strip_v)
    pltpu.sync_copy(strip_v, out_hbm.at[c, pl.ds(s * DSTRIPE, DSTRIPE)])


@functools.partial(
    pl.kernel,
    out_type=jax.ShapeDtypeStruct((NC, N, F), jnp.float32),
    mesh=_mesh,
    compiler_params=pltpu.CompilerParams(use_tc_tiling_on_sc=False, needs_layout_passes=False),
    scratch_types=[
        pltpu.VMEM((NJ1 // 4, C1), jnp.int32),
        pltpu.VMEM((NJ1 // 4, C1), jnp.int32),
        pltpu.VMEM((C1, F), jnp.float32),
        pltpu.VMEM((C1, F), jnp.float32),
        pltpu.VMEM_SHARED((N, F), jnp.float32),
        pltpu.SemaphoreType.DMA,
        pltpu.SemaphoreType.DMA,
    ],
)
def _conv1_kernel(y_hbm, src_hbm, dst_hbm, out_hbm, srcl, dstl, rows0, rows1,
                  agg_sh, sem0, sem1):
    c = lax.axis_index("c")
    s = lax.axis_index("s")
    w = _wid()
    nsup = NJ1 // 4

    def fill(i, _):
        rows0[i // (F // L), pl.ds((i % (F // L)) * L, L)] = (
            jnp.zeros((L,), jnp.float32))
        return 0
    lax.fori_loop(0, C1 * (F // L), fill, 0)
    for t in range(RPT // C1):
        pltpu.sync_copy(rows0, agg_sh.at[pl.ds(s * RPT + t * C1, C1)])
    plsc.subcore_barrier()

    def sup(so, _):
        pltpu.sync_copy(src_hbm.at[w, pl.ds(so * nsup, nsup)], srcl)
        pltpu.sync_copy(dst_hbm.at[w, pl.ds(so * nsup, nsup)], dstl)
        pltpu.async_copy(y_hbm.at[srcl.at[0]], rows0, sem0)

        def pair(jj, _):
            j0 = 2 * jj
            pltpu.make_async_copy(y_hbm.at[srcl.at[j0]], rows0, sem0).wait()
            pltpu.async_copy(y_hbm.at[srcl.at[j0 + 1]], rows1, sem1)
            pltpu.sync_copy(rows0, agg_sh.at[dstl.at[j0]], add=True)
            pltpu.make_async_copy(y_hbm.at[srcl.at[j0 + 1]], rows1,
                                  sem1).wait()

            @pl.when(jj + 1 < nsup // 2)
            def _():
                pltpu.async_copy(y_hbm.at[srcl.at[j0 + 2]], rows0, sem0)
            pltpu.sync_copy(rows1, agg_sh.at[dstl.at[j0 + 1]], add=True)
            return 0
        lax.fori_loop(0, nsup // 2, pair, 0)
        return 0
    lax.fori_loop(0, 4, sup, 0)

    plsc.subcore_barrier()
    for t in range(RPT // C1):
        pltpu.sync_copy(agg_sh.at[pl.ds(s * RPT + t * C1, C1)], rows0)
        pltpu.sync_copy(rows0, out_hbm.at[c, pl.ds(s * RPT + t * C1, C1)])


@functools.partial(
    pl.kernel,
    out_type=jax.ShapeDtypeStruct((NC, N, F), jnp.float32),
    mesh=_mesh,
    compiler_params=pltpu.CompilerParams(use_tc_tiling_on_sc=False, needs_layout_passes=False),
    scratch_types=[
        pltpu.VMEM((NJ2 // 5, C2), jnp.int32),
        pltpu.VMEM((NJ2 // 5, C2), jnp.int32),
        pltpu.VMEM((C2, F), jnp.float32),
        pltpu.VMEM((C2, F), jnp.float32),
        pltpu.VMEM((C2, F), jnp.float32),
        pltpu.VMEM((N,), jnp.float32),
        pltpu.VMEM((L,), jnp.float32),
        pltpu.VMEM_SHARED((N, F), jnp.float32),
        pltpu.SemaphoreType.DMA,
        pltpu.SemaphoreType.DMA,
    ],
)
def _conv2_kernel(h_hbm, src_hbm, dst_hbm, isd_hbm, gam_hbm, out_hbm,
                  srcl, dstl, sbuf, dbuf, mbuf, isd_v, w_v, agg_sh, sem,
                  sem2):
    c = lax.axis_index("c")
    s = lax.axis_index("s")
    w = _wid()
    nsup = NJ2 // 5

    def fill(i, _):
        sbuf[i // (F // L), pl.ds((i % (F // L)) * L, L)] = (
            jnp.zeros((L,), jnp.float32))
        return 0
    lax.fori_loop(0, C2 * (F // L), fill, 0)
    for t in range(RPT // C2):
        pltpu.sync_copy(sbuf, agg_sh.at[pl.ds(s * RPT + t * C2, C2)])
    pltpu.sync_copy(sbuf.at[pl.ds(0, RPT - (RPT // C2) * C2)],
                    agg_sh.at[pl.ds(s * RPT + (RPT // C2) * C2,
                                    RPT - (RPT // C2) * C2)])
    plsc.subcore_barrier()

    pltpu.sync_copy(isd_hbm, isd_v)
    pltpu.sync_copy(gam_hbm, w_v)
    gam = w_v[...]

    def sup(so, _):
        @pl.when(so > 0)
        def _():
            pltpu.make_async_copy(mbuf, agg_sh.at[dstl.at[nsup - 1]],
                                  sem2).wait()
        pltpu.sync_copy(src_hbm.at[w, pl.ds(so * nsup, nsup)], srcl)
        pltpu.sync_copy(dst_hbm.at[w, pl.ds(so * nsup, nsup)], dstl)

        def chunk(j, _):
            cp_s = pltpu.async_copy(h_hbm.at[srcl.at[j]], sbuf, sem)
            cp_d = pltpu.async_copy(h_hbm.at[dstl.at[j]], dbuf, sem)

            @pl.when(j > 0)
            def _():
                pltpu.make_async_copy(mbuf, agg_sh.at[dstl.at[j]],
                                      sem2).wait()
            cp_s.wait()
            cp_d.wait()

            @plsc.parallel_loop(0, NG2, 1, unroll=2)
            def group(g):
                src_v = srcl[j, pl.ds(g * L, L)]
                isd_s = plsc.load_gather(isd_v, [src_v])
                for el in range(L):
                    e = g * L + el
                    sv = [sbuf[e, pl.ds(r * L, L)] for r in range(F // L)]
                    sq = [None] * (F // L)
                    for r in range(F // L):
                        d = sv[r] - dbuf[e, pl.ds(r * L, L)]
                        sq[r] = d * d
                    acc = (((sq[0] + sq[1]) + (sq[2] + sq[3]))
                           + ((sq[4] + sq[5]) + (sq[6] + sq[7])))
                    d2 = lax.broadcast(jnp.sum(acc), (L,))
                    wb = jnp.exp(-gam * d2) * lax.broadcast(isd_s[el], (L,))
                    for r in range(F // L):
                        mbuf[e, pl.ds(r * L, L)] = sv[r] * wb
            pltpu.async_copy(mbuf, agg_sh.at[dstl.at[j]], sem2, add=True)
            return 0
        lax.fori_loop(0, nsup, chunk, 0)
        return 0
    lax.fori_loop(0, 5, sup, 0)

    pltpu.make_async_copy(mbuf, agg_sh.at[dstl.at[nsup - 1]], sem2).wait()
    plsc.subcore_barrier()
    for t in range(RPT // C2):
        pltpu.sync_copy(agg_sh.at[pl.ds(s * RPT + t * C2, C2)], sbuf)
        pltpu.sync_copy(sbuf, out_hbm.at[c, pl.ds(s * RPT + t * C2, C2)])
    rem = RPT - (RPT // C2) * C2
    pltpu.sync_copy(agg_sh.at[pl.ds(s * RPT + (RPT // C2) * C2, rem)],
                    sbuf.at[pl.ds(0, rem)])
    pltpu.sync_copy(sbuf.at[pl.ds(0, rem)],
                    out_hbm.at[c, pl.ds(s * RPT + (RPT // C2) * C2, rem)])


@functools.partial(
    pl.kernel,
    out_type=jax.ShapeDtypeStruct((NB, F), jnp.float32),
    mesh=_mesh,
    compiler_params=pltpu.CompilerParams(use_tc_tiling_on_sc=False, needs_layout_passes=False),
    scratch_types=[
        pltpu.VMEM((NB // NW,), jnp.int32),
        pltpu.VMEM((NB // NW, F), jnp.float32),
        pltpu.SemaphoreType.DMA,
    ],
)
def _gather_kernel(xh_hbm, idx_hbm, out_hbm, idx_v, rows, sem):
    w = _wid()
    bpw = NB // NW
    pltpu.sync_copy(idx_hbm.at[w], idx_v)
    pltpu.async_copy(xh_hbm.at[idx_v], rows, sem).wait()
    pltpu.sync_copy(rows, out_hbm.at[pl.ds(w * bpw, bpw)])


def _pre_body(dp_ref, x_ref, isd_ref, y_ref):
    d = dp_ref[:, 0:1] + dp_ref[:, 1:2] + 1.0
    isd = lax.rsqrt(d)
    isd_ref[...] = isd
    y_ref[...] = x_ref[...] * isd


def _dense1_body(x_ref, ap_ref, isd_ref, k2_ref, w_ref, b_ref, bnw_ref,
                 bnb_ref, h1_ref, helm_ref):
    k2 = k2_ref[0, 0]
    agg = isd_ref[...] * (ap_ref[0] + ap_ref[1])
    x = x_ref[...]
    diff = agg - k2 * x
    helm_ref[...] = jnp.full((1, 1), 0.5, jnp.float32) * jnp.mean(
        diff * diff, keepdims=True)
    h = k2 * x + agg
    o = jnp.dot(h, w_ref[...], preferred_element_type=jnp.float32) + b_ref[...]
    m = jnp.mean(o, axis=0, keepdims=True)
    cdev = o - m
    v = jnp.mean(cdev * cdev, axis=0, keepdims=True)
    h1_ref[...] = jnp.tanh(cdev / jnp.sqrt(v + 1e-5) * bnw_ref[...]
                           + bnb_ref[...])


def _dense2_body(x_ref, ap_ref, isd_ref, k2_ref, w_ref, b_ref, bnw_ref,
                 bnb_ref, out_ref):
    k2 = k2_ref[0, 0]
    agg = isd_ref[...] * (ap_ref[0] + ap_ref[1])
    h = k2 * x_ref[...] + agg
    o = jnp.dot(h, w_ref[...], preferred_element_type=jnp.float32) + b_ref[...]
    m = jnp.mean(o, axis=0, keepdims=True)
    cdev = o - m
    v = jnp.mean(cdev * cdev, axis=0, keepdims=True)
    xh = jnp.tanh(cdev / jnp.sqrt(v + 1e-5) * bnw_ref[...] + bnb_ref[...])
    m2 = jnp.max(xh, axis=1, keepdims=True)
    ex = jnp.exp(xh - m2)
    lse = jnp.log(jnp.sum(ex, axis=1, keepdims=True))
    out_ref[...] = xh - m2 - lse


def kernel(features, edge_index, batch_nodes, device, W1, b1, k2_1, gamma_1,
           bn1_w, bn1_b, W2, b2, k2_2, gamma_2, bn2_w, bn2_b):
    x = features
    src = edge_index[0]
    dst = edge_index[1]
    src3a = src.reshape(NW, NJ1, C1)
    dst3a = dst.reshape(NW, NJ1, C1)
    src3b = src.reshape(NW, NJ2, C2)
    dst3b = dst.reshape(NW, NJ2, C2)

    degp = _deg_kernel(dst3a)
    dpT = degp[:, :N].T

    isd, y = pl.pallas_call(
        _pre_body,
        out_shape=[jax.ShapeDtypeStruct((N, 1), jnp.float32),
                   jax.ShapeDtypeStruct((N, F), jnp.float32)],
    )(dpT, x)

    aggp1 = _conv1_kernel(y, src3a, dst3a)

    h1, helm = pl.pallas_call(
        _dense1_body,
        out_shape=[jax.ShapeDtypeStruct((N, F), jnp.float32),
                   jax.ShapeDtypeStruct((1, 1), jnp.float32)],
    )(x, aggp1, isd, k2_1.reshape(1, 1), W1, b1.reshape(1, F),
      bn1_w.reshape(1, F), bn1_b.reshape(1, F))

    gam16 = jnp.full((L,), gamma_2, jnp.float32)
    aggp2 = _conv2_kernel(h1, src3b, dst3b, isd.reshape(N), gam16)

    xh = pl.pallas_call(
        _dense2_body,
        out_shape=jax.ShapeDtypeStruct((N, F), jnp.float32),
    )(h1, aggp2, isd, k2_2.reshape(1, 1), W2, b2.reshape(1, F),
      bn2_w.reshape(1, F), bn2_b.reshape(1, F))

    logits = _gather_kernel(xh, batch_nodes.reshape(NW, NB // NW))

    return logits, helm.reshape(())
